# Initial kernel scaffold; baseline (speedup 1.0000x reference)
import jax
import jax.numpy as jnp
from jax.experimental import pallas as pl


def kernel(latent, action_type, emb_table, rms_weight, W1, b1, W2, b2):
    raise NotImplementedError("write your pallas kernel here")


# speedup vs baseline: 1.2821x; 1.2821x over previous
import functools

import jax
import jax.numpy as jnp
from jax.experimental import pallas as pl
from jax.experimental.pallas import tpu as pltpu

EPS = 1e-06
D_LAT = 2048
D_EMB = 64
D_IN = D_LAT + D_EMB
TM = 512


def _mlp_kernel(act_ref, lat_ref, emb_ref, wl_ref, we_ref, w1a_ref, w1b_ref,
                b1_ref, w2_ref, b2_ref, out_ref):
    del act_ref
    x = lat_ref[0]
    emb = emb_ref[0]
    sumsq = jnp.sum(x * x, axis=-1, keepdims=True) + jnp.sum(emb * emb)
    scale = jax.lax.rsqrt(sumsq * (1.0 / D_IN) + EPS)
    y = (x * wl_ref[...]).astype(jnp.bfloat16)
    pre = jnp.dot(y, w1a_ref[...], preferred_element_type=jnp.float32)
    ev = jnp.dot((emb * we_ref[...]).astype(jnp.bfloat16), w1b_ref[...],
                 preferred_element_type=jnp.float32)
    h = scale * (pre + ev) + b1_ref[...]
    h = jnp.maximum(h, 0.0).astype(jnp.bfloat16)
    out = jnp.dot(h, w2_ref[...], preferred_element_type=jnp.float32)
    out_ref[0] = out + b2_ref[...]


@jax.jit
def kernel(latent, action_type, emb_table, rms_weight, W1, b1, W2, b2):
    B, T, _ = latent.shape
    HID = W1.shape[1]
    MAX_ACT = W2.shape[1]

    act = action_type.astype(jnp.int32)
    emb3 = emb_table.reshape(emb_table.shape[0], 1, D_EMB)
    wl = rms_weight[:D_LAT].reshape(1, D_LAT)
    we = rms_weight[D_LAT:].reshape(1, D_EMB)
    w1a = W1[:D_LAT].astype(jnp.bfloat16)
    w1b = W1[D_LAT:].astype(jnp.bfloat16)
    w2 = W2.astype(jnp.bfloat16)
    b1r = b1.reshape(1, HID)
    b2r = b2.reshape(1, MAX_ACT)

    grid = (B, T // TM)
    grid_spec = pltpu.PrefetchScalarGridSpec(
        num_scalar_prefetch=1,
        grid=grid,
        in_specs=[
            pl.BlockSpec((1, TM, D_LAT), lambda b, i, act: (b, i, 0)),
            pl.BlockSpec((1, 1, D_EMB), lambda b, i, act: (act[b], 0, 0)),
            pl.BlockSpec((1, D_LAT), lambda b, i, act: (0, 0)),
            pl.BlockSpec((1, D_EMB), lambda b, i, act: (0, 0)),
            pl.BlockSpec((D_LAT, HID), lambda b, i, act: (0, 0)),
            pl.BlockSpec((D_EMB, HID), lambda b, i, act: (0, 0)),
            pl.BlockSpec((1, HID), lambda b, i, act: (0, 0)),
            pl.BlockSpec((HID, MAX_ACT), lambda b, i, act: (0, 0)),
            pl.BlockSpec((1, MAX_ACT), lambda b, i, act: (0, 0)),
        ],
        out_specs=pl.BlockSpec((1, TM, MAX_ACT), lambda b, i, act: (b, i, 0)),
    )
    return pl.pallas_call(
        _mlp_kernel,
        grid_spec=grid_spec,
        out_shape=jax.ShapeDtypeStruct((B, T, MAX_ACT), jnp.float32),
        compiler_params=pltpu.CompilerParams(
            dimension_semantics=("parallel", "parallel"),
        ),
    )(act, latent, emb3, wl, we, w1a, w1b, b1r, w2, b2r)


# speedup vs baseline: 1.2859x; 1.0030x over previous
import functools

import jax
import jax.numpy as jnp
from jax.experimental import pallas as pl
from jax.experimental.pallas import tpu as pltpu

EPS = 1e-06
D_LAT = 2048
D_EMB = 64
D_IN = D_LAT + D_EMB
TM = 512


def _mlp_kernel(act_ref, lat_ref, emb_ref, w1a_ref, w1b_ref,
                b1_ref, w2_ref, b2_ref, out_ref):
    del act_ref
    x = lat_ref[0]
    emb = emb_ref[0]
    sumsq = jnp.sum(x * x, axis=-1, keepdims=True) + jnp.sum(emb * emb)
    scale = jax.lax.rsqrt(sumsq * (1.0 / D_IN) + EPS)
    pre = jnp.dot(x.astype(jnp.bfloat16), w1a_ref[...],
                  preferred_element_type=jnp.float32)
    ev = jnp.dot(emb.astype(jnp.bfloat16), w1b_ref[...],
                 preferred_element_type=jnp.float32)
    h = scale * (pre + ev) + b1_ref[...]
    h = jnp.maximum(h, 0.0).astype(jnp.bfloat16)
    out = jnp.dot(h, w2_ref[...], preferred_element_type=jnp.float32)
    out_ref[0] = out + b2_ref[...]


@jax.jit
def kernel(latent, action_type, emb_table, rms_weight, W1, b1, W2, b2):
    B, T, _ = latent.shape
    HID = W1.shape[1]
    MAX_ACT = W2.shape[1]

    act = action_type.astype(jnp.int32)
    emb3 = emb_table.reshape(emb_table.shape[0], 1, D_EMB)
    w1a = (W1[:D_LAT] * rms_weight[:D_LAT, None]).astype(jnp.bfloat16)
    w1b = (W1[D_LAT:] * rms_weight[D_LAT:, None]).astype(jnp.bfloat16)
    w2 = W2.astype(jnp.bfloat16)
    b1r = b1.reshape(1, HID)
    b2r = b2.reshape(1, MAX_ACT)

    grid = (B, T // TM)
    grid_spec = pltpu.PrefetchScalarGridSpec(
        num_scalar_prefetch=1,
        grid=grid,
        in_specs=[
            pl.BlockSpec((1, TM, D_LAT), lambda b, i, act: (b, i, 0)),
            pl.BlockSpec((1, 1, D_EMB), lambda b, i, act: (act[b], 0, 0)),
            pl.BlockSpec((D_LAT, HID), lambda b, i, act: (0, 0)),
            pl.BlockSpec((D_EMB, HID), lambda b, i, act: (0, 0)),
            pl.BlockSpec((1, HID), lambda b, i, act: (0, 0)),
            pl.BlockSpec((HID, MAX_ACT), lambda b, i, act: (0, 0)),
            pl.BlockSpec((1, MAX_ACT), lambda b, i, act: (0, 0)),
        ],
        out_specs=pl.BlockSpec((1, TM, MAX_ACT), lambda b, i, act: (b, i, 0)),
    )
    return pl.pallas_call(
        _mlp_kernel,
        grid_spec=grid_spec,
        out_shape=jax.ShapeDtypeStruct((B, T, MAX_ACT), jnp.float32),
        compiler_params=pltpu.CompilerParams(
            dimension_semantics=("parallel", "parallel"),
        ),
    )(act, latent, emb3, w1a, w1b, b1r, w2, b2r)


# speedup vs baseline: 1.4393x; 1.1192x over previous
import functools

import jax
import jax.numpy as jnp
from jax.experimental import pallas as pl
from jax.experimental.pallas import tpu as pltpu

EPS = 1e-06
D_LAT = 2048
D_EMB = 64
D_IN = D_LAT + D_EMB
TM = 1024


def _mlp_kernel(act_ref, lat_ref, emb_ref, w1a_ref, w1b_ref,
                b1_ref, w2_ref, b2_ref, out_ref):
    del act_ref
    x = lat_ref[0]
    emb = emb_ref[0]
    sumsq = jnp.sum(x * x, axis=-1, keepdims=True) + jnp.sum(emb * emb)
    scale = jax.lax.rsqrt(sumsq * (1.0 / D_IN) + EPS)
    pre = jnp.dot(x.astype(jnp.bfloat16), w1a_ref[...],
                  preferred_element_type=jnp.float32)
    ev = jnp.dot(emb.astype(jnp.bfloat16), w1b_ref[...],
                 preferred_element_type=jnp.float32)
    h = scale * (pre + ev) + b1_ref[...]
    h = jnp.maximum(h, 0.0).astype(jnp.bfloat16)
    out = jnp.dot(h, w2_ref[...], preferred_element_type=jnp.float32)
    out_ref[0] = out + b2_ref[...]


@jax.jit
def kernel(latent, action_type, emb_table, rms_weight, W1, b1, W2, b2):
    B, T, _ = latent.shape
    HID = W1.shape[1]
    MAX_ACT = W2.shape[1]

    act = action_type.astype(jnp.int32)
    emb3 = emb_table.reshape(emb_table.shape[0], 1, D_EMB)
    w1a = (W1[:D_LAT] * rms_weight[:D_LAT, None]).astype(jnp.bfloat16)
    w1b = (W1[D_LAT:] * rms_weight[D_LAT:, None]).astype(jnp.bfloat16)
    w2 = W2.astype(jnp.bfloat16)
    b1r = b1.reshape(1, HID)
    b2r = b2.reshape(1, MAX_ACT)

    grid = (B, T // TM)
    grid_spec = pltpu.PrefetchScalarGridSpec(
        num_scalar_prefetch=1,
        grid=grid,
        in_specs=[
            pl.BlockSpec((1, TM, D_LAT), lambda b, i, act: (b, i, 0)),
            pl.BlockSpec((1, 1, D_EMB), lambda b, i, act: (act[b], 0, 0)),
            pl.BlockSpec((D_LAT, HID), lambda b, i, act: (0, 0)),
            pl.BlockSpec((D_EMB, HID), lambda b, i, act: (0, 0)),
            pl.BlockSpec((1, HID), lambda b, i, act: (0, 0)),
            pl.BlockSpec((HID, MAX_ACT), lambda b, i, act: (0, 0)),
            pl.BlockSpec((1, MAX_ACT), lambda b, i, act: (0, 0)),
        ],
        out_specs=pl.BlockSpec((1, TM, MAX_ACT), lambda b, i, act: (b, i, 0)),
    )
    return pl.pallas_call(
        _mlp_kernel,
        grid_spec=grid_spec,
        out_shape=jax.ShapeDtypeStruct((B, T, MAX_ACT), jnp.float32),
        compiler_params=pltpu.CompilerParams(
            dimension_semantics=("parallel", "parallel"),
        ),
    )(act, latent, emb3, w1a, w1b, b1r, w2, b2r)


# speedup vs baseline: 1.5028x; 1.0442x over previous
import functools

import jax
import jax.numpy as jnp
from jax.experimental import pallas as pl
from jax.experimental.pallas import tpu as pltpu

EPS = 1e-06
D_LAT = 2048
D_EMB = 64
D_IN = D_LAT + D_EMB
TM = 2048


def _mlp_kernel(act_ref, lat_ref, emb_ref, w1a_ref, w1b_ref,
                b1_ref, w2_ref, b2_ref, out_ref):
    del act_ref
    x = lat_ref[0]
    emb = emb_ref[0]
    sumsq = jnp.sum(x * x, axis=-1, keepdims=True) + jnp.sum(emb * emb)
    scale = jax.lax.rsqrt(sumsq * (1.0 / D_IN) + EPS)
    pre = jnp.dot(x.astype(jnp.bfloat16), w1a_ref[...],
                  preferred_element_type=jnp.float32)
    ev = jnp.dot(emb.astype(jnp.bfloat16), w1b_ref[...],
                 preferred_element_type=jnp.float32)
    h = scale * (pre + ev) + b1_ref[...]
    h = jnp.maximum(h, 0.0).astype(jnp.bfloat16)
    out = jnp.dot(h, w2_ref[...], preferred_element_type=jnp.float32)
    out_ref[0] = out + b2_ref[...]


@jax.jit
def kernel(latent, action_type, emb_table, rms_weight, W1, b1, W2, b2):
    B, T, _ = latent.shape
    HID = W1.shape[1]
    MAX_ACT = W2.shape[1]

    act = action_type.astype(jnp.int32)
    emb3 = emb_table.reshape(emb_table.shape[0], 1, D_EMB)
    w1a = (W1[:D_LAT] * rms_weight[:D_LAT, None]).astype(jnp.bfloat16)
    w1b = (W1[D_LAT:] * rms_weight[D_LAT:, None]).astype(jnp.bfloat16)
    w2 = W2.astype(jnp.bfloat16)
    b1r = b1.reshape(1, HID)
    b2r = b2.reshape(1, MAX_ACT)

    grid = (B, T // TM)
    grid_spec = pltpu.PrefetchScalarGridSpec(
        num_scalar_prefetch=1,
        grid=grid,
        in_specs=[
            pl.BlockSpec((1, TM, D_LAT), lambda b, i, act: (b, i, 0)),
            pl.BlockSpec((1, 1, D_EMB), lambda b, i, act: (act[b], 0, 0)),
            pl.BlockSpec((D_LAT, HID), lambda b, i, act: (0, 0)),
            pl.BlockSpec((D_EMB, HID), lambda b, i, act: (0, 0)),
            pl.BlockSpec((1, HID), lambda b, i, act: (0, 0)),
            pl.BlockSpec((HID, MAX_ACT), lambda b, i, act: (0, 0)),
            pl.BlockSpec((1, MAX_ACT), lambda b, i, act: (0, 0)),
        ],
        out_specs=pl.BlockSpec((1, TM, MAX_ACT), lambda b, i, act: (b, i, 0)),
    )
    return pl.pallas_call(
        _mlp_kernel,
        grid_spec=grid_spec,
        out_shape=jax.ShapeDtypeStruct((B, T, MAX_ACT), jnp.float32),
        compiler_params=pltpu.CompilerParams(
            dimension_semantics=("parallel", "parallel"),
        ),
    )(act, latent, emb3, w1a, w1b, b1r, w2, b2r)
